# baseline (speedup 1.0000x reference)
import functools

import jax
import jax.numpy as jnp
from jax import lax
from jax.experimental import pallas as pl
from jax.experimental.pallas import tpu as pltpu
from jax.experimental.pallas import tpu_sc as plsc

B, C, H, W = 4, 192, 224, 224
PLANES = B * C
NC, NS = 2, 16
NW = NC * NS
NG = PLANES // NW

_MESH = plsc.VectorSubcoreMesh(core_axis_name="c", subcore_axis_name="s")


@functools.partial(
    pl.kernel,
    out_type=jax.ShapeDtypeStruct((PLANES, H, W), jnp.float32),
    mesh=_MESH,
    scratch_types=[
        pltpu.VMEM((32,), jnp.int32),
        pltpu.VMEM_SHARED((NS, 2, 1, H, W), jnp.float32),
        pltpu.SemaphoreType.DMA,
        pltpu.SemaphoreType.DMA,
        pltpu.SemaphoreType.DMA,
        pltpu.SemaphoreType.DMA,
    ],
)
def _permute_planes(in_hbm, idx_hbm, out_hbm, idx_v, shared, g0, g1, s0, s1):
    wid = lax.axis_index("s") * NC + lax.axis_index("c")
    sid = lax.axis_index("s")
    base = wid * NG
    pltpu.sync_copy(idx_hbm.at[wid], idx_v)
    bufs = (shared.at[sid, 0], shared.at[sid, 1])
    gsem = (g0, g1)
    ssem = (s0, s1)

    lo, hi = idx_v[pl.ds(0, 16)], idx_v[pl.ds(16, 16)]

    def src(g):
        return lo[g] if g < 16 else hi[g - 16]

    COLS = ((0, 128), (128, 96))

    def gather(g, b):
        for o, n in COLS:
            pltpu.async_copy(
                in_hbm.at[pl.ds(src(g), 1), slice(None), pl.ds(o, n)],
                bufs[b].at[:, :, pl.ds(o, n)], gsem[b])

    def gather_wait(b):
        for o, n in COLS:
            pltpu.make_async_copy(
                in_hbm.at[pl.ds(0, 1), slice(None), pl.ds(o, n)],
                bufs[b].at[:, :, pl.ds(o, n)], gsem[b]).wait()

    def scatter(g, b):
        for o, n in COLS:
            pltpu.async_copy(
                bufs[b].at[:, :, pl.ds(o, n)],
                out_hbm.at[pl.ds(base + g, 1), slice(None), pl.ds(o, n)],
                ssem[b])

    def scatter_wait(g, b):
        for o, n in COLS:
            pltpu.make_async_copy(
                bufs[b].at[:, :, pl.ds(o, n)],
                out_hbm.at[pl.ds(base + g, 1), slice(None), pl.ds(o, n)],
                ssem[b]).wait()

    for b in range(2):
        gather(b, b)

    for g in range(NG):
        b = g & 1
        gather_wait(b)
        scatter(g, b)
        if g + 2 < NG:
            scatter_wait(g, b)
            gather(g + 2, b)

    for g in (NG - 2, NG - 1):
        scatter_wait(g, g & 1)


def kernel(input, indices):
    src_plane = (jnp.arange(B, dtype=jnp.int32)[:, None] * C
                 + indices[None, :].astype(jnp.int32))
    idx = jnp.pad(src_plane.reshape(NW, NG), ((0, 0), (0, 32 - NG)))
    out = _permute_planes(input.reshape(PLANES, H, W), idx)
    return out.reshape(input.shape), 0.0
